# baseline (speedup 1.0000x reference)
import functools

import jax
import jax.numpy as jnp
import numpy as np
from jax import lax
from jax.experimental import pallas as pl
from jax.experimental.pallas import tpu as pltpu
from jax.experimental.pallas import tpu_sc as plsc

D_MODEL = 1024
MAX_LEN = 4096

NC = 2
NS = 16
NW = NC * NS
LANES = 16
CH = 16
NBUF = 6
LOOKAHEAD = 4


def _positional_encoding(max_len, d_model):
    pos = np.arange(max_len, dtype=np.float32)[:, None]
    i = np.arange(0, d_model, 2, dtype=np.float32)[None, :]
    angle = pos / np.power(10000.0, i / float(d_model))
    pe = np.zeros((max_len, d_model), dtype=np.float32)
    pe[:, 0::2] = np.sin(angle)
    pe[:, 1::2] = np.cos(angle)
    return pe


@functools.lru_cache(maxsize=None)
def _pe_const_packed(seq_len, d_model):
    import ml_dtypes
    pe = _positional_encoding(MAX_LEN, d_model)[:seq_len]
    pe = pe.astype(ml_dtypes.bfloat16).view(np.uint16)
    blocks = pe.reshape(seq_len, d_model // 32, 2, 16)
    words = blocks[:, :, 0, :].astype(np.uint32) \
        | (blocks[:, :, 1, :].astype(np.uint32) << 16)
    return np.ascontiguousarray(
        words.reshape(seq_len, d_model // 2).view(np.int32))


@functools.lru_cache(maxsize=None)
def _build(B, S, V, D):
    PPW = S // NW
    NPC = PPW // CH
    NCHUNK = NPC * B
    VECS = D // LANES
    assert S % NW == 0 and PPW % CH == 0 and D % (2 * LANES) == 0
    assert (VECS & (VECS - 1)) == 0

    mesh = plsc.VectorSubcoreMesh(
        core_axis_name="c", subcore_axis_name="s", num_cores=NC,
        num_subcores=NS)

    @functools.partial(
        pl.kernel,
        out_type=jax.ShapeDtypeStruct((B * S, D), jnp.float32),
        mesh=mesh,
        scratch_types=[
            pltpu.VMEM((B, PPW), jnp.int32),
            [pltpu.VMEM((CH, D), jnp.float32)] * NBUF,
            [pltpu.VMEM((CH, D // 2), jnp.int32)] * 2,
            pltpu.SemaphoreType.DMA((NBUF,)),
            pltpu.SemaphoreType.DMA((NBUF,)),
            pltpu.SemaphoreType.DMA((2,)),
        ],
    )
    def emb_kernel(x_hbm, table_hbm, pe_hbm, out_hbm, idx_v, rows, pes,
                   sem_g, sem_w, sem_pe):
        w = lax.axis_index("s") * NC + lax.axis_index("c")
        pos0 = w * PPW

        for b in range(B):
            pltpu.sync_copy(x_hbm.at[b, pl.ds(pos0, PPW)], idx_v.at[b])

        def issue_gather(g):
            jj, b = divmod(g, B)
            return pltpu.async_copy(
                table_hbm.at[idx_v.at[b, pl.ds(jj * CH, CH)]],
                rows[g % NBUF], sem_g.at[g % NBUF])

        def issue_pe(jj):
            return pltpu.async_copy(
                pe_hbm.at[pl.ds(pos0 + jj * CH, CH)],
                pes[jj % 2], sem_pe.at[jj % 2])

        def issue_write(g):
            jj, b = divmod(g, B)
            flat0 = b * S + pos0 + jj * CH
            return pltpu.async_copy(rows[g % NBUF],
                                    out_hbm.at[pl.ds(flat0, CH)],
                                    sem_w.at[g % NBUF])

        pdesc = {}
        for jj in range(min(2, NPC)):
            pdesc[jj] = issue_pe(jj)
        gdesc = {}
        for g in range(min(LOOKAHEAD, NCHUNK)):
            gdesc[g] = issue_gather(g)

        PAIRS = VECS // 2
        wdesc = {}
        for g in range(NCHUNK):
            jj, b = divmod(g, B)
            if g - LOOKAHEAD >= 0:
                wdesc[g - LOOKAHEAD].wait()
            if g + LOOKAHEAD < NCHUNK:
                gdesc[g + LOOKAHEAD] = issue_gather(g + LOOKAHEAD)
            gdesc[g].wait()
            if b == 0:
                pdesc[jj].wait()

            buf = rows[g % NBUF]
            pe_buf = pes[jj % 2]

            @plsc.parallel_loop(0, CH * PAIRS, unroll=16)
            def _(v):
                i = v >> (PAIRS.bit_length() - 1)
                c = (v & (PAIRS - 1)) * 2 * LANES
                w16 = pe_buf[i, pl.ds((v & (PAIRS - 1)) * LANES, LANES)]
                lo = lax.bitcast_convert_type(w16 << 16, jnp.float32)
                hi = lax.bitcast_convert_type(w16 & jnp.int32(-65536),
                                              jnp.float32)
                plsc.addupdate(buf.at[i, pl.ds(c, LANES)], lo)
                plsc.addupdate(buf.at[i, pl.ds(c + LANES, LANES)], hi)

            wdesc[g] = issue_write(g)
            if b == B - 1 and jj + 2 < NPC:
                pdesc[jj + 2] = issue_pe(jj + 2)

        for g in range(max(0, NCHUNK - LOOKAHEAD), NCHUNK):
            wdesc[g].wait()

    return emb_kernel


def kernel(x, table):
    B, S = x.shape
    V, D = table.shape
    pe = _pe_const_packed(S, D)
    out = _build(B, S, V, D)(x, table, pe)
    return out.reshape(B, S, D)


# speedup vs baseline: 1.0573x; 1.0573x over previous
import functools

import jax
import jax.numpy as jnp
import numpy as np
from jax import lax
from jax.experimental import pallas as pl
from jax.experimental.pallas import tpu as pltpu
from jax.experimental.pallas import tpu_sc as plsc

D_MODEL = 1024
MAX_LEN = 4096

NC = 2
NS = 16
NW = NC * NS
LANES = 16
CH = 16
NBUF = 6
LOOKAHEAD = 5


def _positional_encoding(max_len, d_model):
    pos = np.arange(max_len, dtype=np.float32)[:, None]
    i = np.arange(0, d_model, 2, dtype=np.float32)[None, :]
    angle = pos / np.power(10000.0, i / float(d_model))
    pe = np.zeros((max_len, d_model), dtype=np.float32)
    pe[:, 0::2] = np.sin(angle)
    pe[:, 1::2] = np.cos(angle)
    return pe


@functools.lru_cache(maxsize=None)
def _pe_const_packed(seq_len, d_model):
    import ml_dtypes
    pe = _positional_encoding(MAX_LEN, d_model)[:seq_len]
    pe = pe.astype(ml_dtypes.bfloat16).view(np.uint16)
    blocks = pe.reshape(seq_len, d_model // 32, 2, 16)
    words = blocks[:, :, 0, :].astype(np.uint32) \
        | (blocks[:, :, 1, :].astype(np.uint32) << 16)
    return np.ascontiguousarray(
        words.reshape(seq_len, d_model // 2).view(np.int32))


@functools.lru_cache(maxsize=None)
def _build(B, S, V, D):
    PPW = S // NW
    NPC = PPW // CH
    NCHUNK = NPC * B
    VECS = D // LANES
    assert S % NW == 0 and PPW % CH == 0 and D % (2 * LANES) == 0
    assert (VECS & (VECS - 1)) == 0

    mesh = plsc.VectorSubcoreMesh(
        core_axis_name="c", subcore_axis_name="s", num_cores=NC,
        num_subcores=NS)

    @functools.partial(
        pl.kernel,
        out_type=jax.ShapeDtypeStruct((B * S, D), jnp.float32),
        mesh=mesh,
        scratch_types=[
            pltpu.VMEM((B, PPW), jnp.int32),
            [pltpu.VMEM((CH, D), jnp.float32)] * NBUF,
            [pltpu.VMEM((CH, D // 2), jnp.int32)] * 2,
            pltpu.SemaphoreType.DMA((NBUF,)),
            pltpu.SemaphoreType.DMA((NBUF,)),
            pltpu.SemaphoreType.DMA((2,)),
        ],
    )
    def emb_kernel(x_hbm, table_hbm, pe_hbm, out_hbm, idx_v, rows, pes,
                   sem_g, sem_w, sem_pe):
        w = lax.axis_index("s") * NC + lax.axis_index("c")
        pos0 = w * PPW

        for b in range(B):
            pltpu.sync_copy(x_hbm.at[b, pl.ds(pos0, PPW)], idx_v.at[b])

        def issue_gather(g):
            jj, b = divmod(g, B)
            return pltpu.async_copy(
                table_hbm.at[idx_v.at[b, pl.ds(jj * CH, CH)]],
                rows[g % NBUF], sem_g.at[g % NBUF])

        def issue_pe(jj):
            return pltpu.async_copy(
                pe_hbm.at[pl.ds(pos0 + jj * CH, CH)],
                pes[jj % 2], sem_pe.at[jj % 2])

        def issue_write(g):
            jj, b = divmod(g, B)
            flat0 = b * S + pos0 + jj * CH
            return pltpu.async_copy(rows[g % NBUF],
                                    out_hbm.at[pl.ds(flat0, CH)],
                                    sem_w.at[g % NBUF])

        pdesc = {}
        for jj in range(min(2, NPC)):
            pdesc[jj] = issue_pe(jj)
        gdesc = {}
        for g in range(min(LOOKAHEAD, NCHUNK)):
            gdesc[g] = issue_gather(g)

        PAIRS = VECS // 2
        wdesc = {}
        for g in range(NCHUNK):
            jj, b = divmod(g, B)
            if g - LOOKAHEAD >= 0:
                wdesc[g - LOOKAHEAD].wait()
            if g + LOOKAHEAD < NCHUNK:
                gdesc[g + LOOKAHEAD] = issue_gather(g + LOOKAHEAD)
            gdesc[g].wait()
            if b == 0:
                pdesc[jj].wait()

            buf = rows[g % NBUF]
            pe_buf = pes[jj % 2]

            @plsc.parallel_loop(0, CH * PAIRS, unroll=8)
            def _(v):
                i = v >> (PAIRS.bit_length() - 1)
                c = (v & (PAIRS - 1)) * 2 * LANES
                w16 = pe_buf[i, pl.ds((v & (PAIRS - 1)) * LANES, LANES)]
                lo = lax.bitcast_convert_type(w16 << 16, jnp.float32)
                hi = lax.bitcast_convert_type(w16 & jnp.int32(-65536),
                                              jnp.float32)
                plsc.addupdate(buf.at[i, pl.ds(c, LANES)], lo)
                plsc.addupdate(buf.at[i, pl.ds(c + LANES, LANES)], hi)

            wdesc[g] = issue_write(g)
            if b == B - 1 and jj + 2 < NPC:
                pdesc[jj + 2] = issue_pe(jj + 2)

        for g in range(max(0, NCHUNK - LOOKAHEAD), NCHUNK):
            wdesc[g].wait()

    return emb_kernel


def kernel(x, table):
    B, S = x.shape
    V, D = table.shape
    pe = _pe_const_packed(S, D)
    out = _build(B, S, V, D)(x, table, pe)
    return out.reshape(B, S, D)
